# baseline (speedup 1.0000x reference)
import functools

import jax
import jax.numpy as jnp
from jax import lax
from jax.experimental import pallas as pl
from jax.experimental.pallas import tpu as pltpu
from jax.experimental.pallas import tpu_sc as plsc

DEPTH = 5
NC, NS = 2, 16
NW = NC * NS
MAX_IN = 4
_MASK_HI = -65536


def _pack_bf16(x):
    half = x.shape[1] // 2
    xb = x.astype(jnp.bfloat16)
    lo = lax.convert_element_type(
        lax.bitcast_convert_type(xb[:, :half], jnp.uint16), jnp.uint32)
    hi = lax.convert_element_type(
        lax.bitcast_convert_type(xb[:, half:], jnp.uint16), jnp.uint32)
    return lax.bitcast_convert_type(lo | (hi << 16), jnp.int32)


def _unpack_f32(w):
    lo = lax.bitcast_convert_type(w << 16, jnp.float32)
    hi = lax.bitcast_convert_type(w & _MASK_HI, jnp.float32)
    return lo, hi


def _sum_slots(g, m):
    return ((g[0 * m:1 * m] + g[1 * m:2 * m])
            + (g[2 * m:3 * m] + g[3 * m:4 * m]))


def _slot_major_idx(idx2d, blk):
    m = idx2d.shape[0]
    return (idx2d.reshape(m // blk, blk, MAX_IN)
            .transpose(0, 2, 1).reshape(-1).astype(jnp.int32))


def _gather_sc(table, idx_flat, rows_pc=128):
    n_rows, width = table.shape
    total_rows = idx_flat.shape[0]
    total_chunks = total_rows // rows_pc
    assert total_rows % rows_pc == 0 and rows_pc <= 128
    mesh = plsc.VectorSubcoreMesh(core_axis_name="c", subcore_axis_name="s",
                                  num_cores=NC, num_subcores=NS)

    @functools.partial(
        pl.kernel,
        out_type=jax.ShapeDtypeStruct((total_rows, width), jnp.int32),
        mesh=mesh,
        scratch_types=[
            pltpu.VMEM((rows_pc,), jnp.int32),
            pltpu.VMEM((rows_pc,), jnp.int32),
            pltpu.VMEM((rows_pc, width), jnp.int32),
            pltpu.VMEM((rows_pc, width), jnp.int32),
            pltpu.SemaphoreType.DMA,
            pltpu.SemaphoreType.DMA,
        ],
    )
    def gather_kernel(table_hbm, idx_hbm, out_hbm, idx0, idx1, rows0, rows1,
                      sem0, sem1):
        wid = lax.axis_index("s") * NC + lax.axis_index("c")
        n_mine = (total_chunks - wid + NW - 1) // NW
        idx_b, rows_b, sem_b = (idx0, idx1), (rows0, rows1), (sem0, sem1)

        def start_gather(i, b):
            ci = wid + i * NW
            pltpu.sync_copy(
                idx_hbm.at[pl.ds(rows_pc * ci, rows_pc)], idx_b[b])
            pltpu.async_copy(table_hbm.at[idx_b[b]], rows_b[b], sem_b[b])

        def finish(i, b):
            ci = wid + i * NW
            pltpu.make_async_copy(table_hbm.at[idx_b[b]], rows_b[b],
                                  sem_b[b]).wait()
            pltpu.sync_copy(rows_b[b],
                            out_hbm.at[pl.ds(ci * rows_pc, rows_pc)])

        @pl.when(n_mine > 0)
        def _prime():
            start_gather(0, 0)

        def pair(p, carry):
            i0, i1 = 2 * p, 2 * p + 1

            @pl.when(i1 < n_mine)
            def _s1():
                start_gather(i1, 1)

            finish(i0, 0)

            @pl.when(i1 + 1 < n_mine)
            def _s0():
                start_gather(i1 + 1, 0)

            @pl.when(i1 < n_mine)
            def _f1():
                finish(i1, 1)

            return carry

        lax.fori_loop(0, (n_mine + 1) // 2, pair, 0)

    return gather_kernel(table, idx_flat)


def _proj_body(x_ref, w_ref, inp_ref, msg_ref):
    acc = jnp.dot(x_ref[...], w_ref[...], preferred_element_type=jnp.float32)
    inp_ref[...] = acc
    msg_ref[...] = _pack_bf16(jnp.maximum(acc, 0.0))


def _step_core(g_ref, inp_ref, wt_ref, wb_ref):
    m = inp_ref.shape[0]
    lo, hi = _unpack_f32(g_ref[...])
    lo_s = _sum_slots(lo, m)
    hi_s = _sum_slots(hi, m)
    return (inp_ref[...]
            + jnp.dot(lo_s, wt_ref[...], preferred_element_type=jnp.float32)
            + jnp.dot(hi_s, wb_ref[...], preferred_element_type=jnp.float32))


def _step_body(g_ref, inp_ref, wt_ref, wb_ref, buf_ref, msg_ref):
    del buf_ref
    h = _step_core(g_ref, inp_ref, wt_ref, wb_ref)
    msg_ref[...] = _pack_bf16(jnp.maximum(h, 0.0))


def _step_last_body(g_ref, inp_ref, wt_ref, wb_ref, buf_ref, h_ref):
    del buf_ref
    h_ref[...] = _pack_bf16(_step_core(g_ref, inp_ref, wt_ref, wb_ref))


def _tail_body(apm, af_ref, msgs_ref, gf_ref, woa_ref, womt_ref, womb_ref,
               bo_ref, w1g_ref, w1m_ref, b1_ref, w2_ref, b2_ref, w3t_ref,
               out_ref):
    n_atoms_blk = af_ref.shape[0]
    lo, hi = _unpack_f32(msgs_ref[...])
    lo_s = _sum_slots(lo, n_atoms_blk)
    hi_s = _sum_slots(hi, n_atoms_blk)
    hidden = jnp.maximum(
        jnp.dot(af_ref[...], woa_ref[...], preferred_element_type=jnp.float32)
        + jnp.dot(lo_s, womt_ref[...], preferred_element_type=jnp.float32)
        + jnp.dot(hi_s, womb_ref[...], preferred_element_type=jnp.float32)
        + bo_ref[...], 0.0)
    hid = hidden.shape[1]
    mols = n_atoms_blk // apm
    mol = jnp.mean(hidden.reshape(mols, apm, hid), axis=1)
    h1 = jnp.maximum(
        jnp.dot(mol, w1m_ref[...], preferred_element_type=jnp.float32)
        + jnp.dot(gf_ref[...], w1g_ref[...],
                  preferred_element_type=jnp.float32)
        + b1_ref[...], 0.0)
    h2 = jnp.maximum(
        jnp.dot(h1, w2_ref[...], preferred_element_type=jnp.float32)
        + b2_ref[...], 0.0)
    out_ref[...] = jnp.sum(h2 * w3t_ref[...], axis=1, keepdims=True)


def kernel(atom_features, f_ini_atoms_bonds, atom_to_incoming_bonds, mapping,
           global_features, W_i, W_h, W_o, b_o, W_ffn1, b_ffn1, W_ffn2,
           b_ffn2, W_ffn3, b_ffn3):
    n_atoms, atom_f = atom_features.shape
    n_bonds, concat_f = f_ini_atoms_bonds.shape
    n_mols, gf_dim = global_features.shape
    hid = W_h.shape[0]
    half = hid // 2
    apm = n_atoms // n_mols

    mb = 800
    inp, message = pl.pallas_call(
        _proj_body,
        grid=(n_bonds // mb,),
        in_specs=[
            pl.BlockSpec((mb, concat_f), lambda i: (i, 0)),
            pl.BlockSpec((concat_f, hid), lambda i: (0, 0)),
        ],
        out_specs=[
            pl.BlockSpec((mb, hid), lambda i: (i, 0)),
            pl.BlockSpec((mb, half), lambda i: (i, 0)),
        ],
        out_shape=[
            jax.ShapeDtypeStruct((n_bonds, hid), jnp.float32),
            jax.ShapeDtypeStruct((n_bonds, half), jnp.int32),
        ],
    )(f_ini_atoms_bonds, W_i)

    sb = 1000
    hb = n_bonds // sb // 2
    rows_half = (n_bonds // 2) * MAX_IN
    map_flat = _slot_major_idx(mapping, sb)
    map_a, map_b = map_flat[:rows_half], map_flat[rows_half:]
    wh_top, wh_bot = W_h[:half], W_h[half:]

    def make_step(body, off):
        return pl.pallas_call(
            body,
            grid=(hb,),
            in_specs=[
                pl.BlockSpec((MAX_IN * sb, half), lambda i: (i, 0)),
                pl.BlockSpec((sb, hid), lambda i: (i + off, 0)),
                pl.BlockSpec((half, hid), lambda i: (0, 0)),
                pl.BlockSpec((half, hid), lambda i: (0, 0)),
                pl.BlockSpec(memory_space=pltpu.MemorySpace.HBM),
            ],
            out_specs=pl.BlockSpec((sb, half), lambda i: (i + off, 0)),
            out_shape=jax.ShapeDtypeStruct((n_bonds, half), jnp.int32),
            input_output_aliases={4: 0},
        )

    step_a, step_b = make_step(_step_body, 0), make_step(_step_body, hb)
    last_a, last_b = (make_step(_step_last_body, 0),
                      make_step(_step_last_body, hb))
    hist = [message]
    for d in range(1, DEPTH - 1):
        cur = hist[-1]
        ga = _gather_sc(cur, map_a, rows_pc=80)
        gb = _gather_sc(cur, map_b, rows_pc=80)
        donor = (jnp.zeros((n_bonds, half), jnp.int32) if d == 1
                 else hist[d - 2])
        buf_a = step_a(ga, inp, wh_top, wh_bot, donor)
        hist.append(step_b(gb, inp, wh_top, wh_bot, buf_a))
    cur = hist[-1]
    ga = _gather_sc(cur, map_a, rows_pc=80)
    gb = _gather_sc(cur, map_b, rows_pc=80)
    buf_a = last_a(ga, inp, wh_top, wh_bot, hist[DEPTH - 3])
    h_message = last_b(gb, inp, wh_top, wh_bot, buf_a)

    mol_blk = 40
    atoms_blk = mol_blk * apm
    atib_flat = _slot_major_idx(atom_to_incoming_bonds, atoms_blk)
    msgs_to_atoms = _gather_sc(h_message, atib_flat, rows_pc=80)

    w_om = W_o[atom_f:]
    out = pl.pallas_call(
        functools.partial(_tail_body, apm),
        grid=(n_mols // mol_blk,),
        in_specs=[
            pl.BlockSpec((atoms_blk, atom_f), lambda i: (i, 0)),
            pl.BlockSpec((MAX_IN * atoms_blk, half), lambda i: (i, 0)),
            pl.BlockSpec((mol_blk, gf_dim), lambda i: (i, 0)),
            pl.BlockSpec((atom_f, hid), lambda i: (0, 0)),
            pl.BlockSpec((half, hid), lambda i: (0, 0)),
            pl.BlockSpec((half, hid), lambda i: (0, 0)),
            pl.BlockSpec((1, hid), lambda i: (0, 0)),
            pl.BlockSpec((gf_dim, hid), lambda i: (0, 0)),
            pl.BlockSpec((hid, hid), lambda i: (0, 0)),
            pl.BlockSpec((1, hid), lambda i: (0, 0)),
            pl.BlockSpec((hid, hid), lambda i: (0, 0)),
            pl.BlockSpec((1, hid), lambda i: (0, 0)),
            pl.BlockSpec((1, hid), lambda i: (0, 0)),
        ],
        out_specs=pl.BlockSpec((mol_blk, 1), lambda i: (i, 0)),
        out_shape=jax.ShapeDtypeStruct((n_mols, 1), jnp.float32),
    )(atom_features, msgs_to_atoms, global_features,
      W_o[:atom_f], w_om[:half], w_om[half:], b_o.reshape(1, hid),
      W_ffn1[hid:], W_ffn1[:hid], b_ffn1.reshape(1, hid),
      W_ffn2, b_ffn2.reshape(1, hid), W_ffn3.reshape(1, hid))
    return out + b_ffn3


# speedup vs baseline: 1.0015x; 1.0015x over previous
import functools

import jax
import jax.numpy as jnp
from jax import lax
from jax.experimental import pallas as pl
from jax.experimental.pallas import tpu as pltpu
from jax.experimental.pallas import tpu_sc as plsc

DEPTH = 5
NC, NS = 2, 16
NW = NC * NS
MAX_IN = 4
_MASK_HI = -65536


def _pack_bf16(x):
    half = x.shape[1] // 2
    xb = x.astype(jnp.bfloat16)
    lo = lax.convert_element_type(
        lax.bitcast_convert_type(xb[:, :half], jnp.uint16), jnp.uint32)
    hi = lax.convert_element_type(
        lax.bitcast_convert_type(xb[:, half:], jnp.uint16), jnp.uint32)
    return lax.bitcast_convert_type(lo | (hi << 16), jnp.int32)


def _unpack_f32(w):
    lo = lax.bitcast_convert_type(w << 16, jnp.float32)
    hi = lax.bitcast_convert_type(w & _MASK_HI, jnp.float32)
    return lo, hi


def _sum_slots(g, m):
    return ((g[0 * m:1 * m] + g[1 * m:2 * m])
            + (g[2 * m:3 * m] + g[3 * m:4 * m]))


def _slot_major_idx(idx2d, blk):
    m = idx2d.shape[0]
    return (idx2d.reshape(m // blk, blk, MAX_IN)
            .transpose(0, 2, 1).reshape(-1).astype(jnp.int32))


def _gather_sc(table, idx_flat, rows_pc=128):
    n_rows, width = table.shape
    total_rows = idx_flat.shape[0]
    total_chunks = total_rows // rows_pc
    assert total_rows % rows_pc == 0 and rows_pc <= 128
    mesh = plsc.VectorSubcoreMesh(core_axis_name="c", subcore_axis_name="s",
                                  num_cores=NC, num_subcores=NS)

    @functools.partial(
        pl.kernel,
        out_type=jax.ShapeDtypeStruct((total_rows, width), jnp.int32),
        mesh=mesh,
        scratch_types=[
            pltpu.VMEM((rows_pc,), jnp.int32),
            pltpu.VMEM((rows_pc,), jnp.int32),
            pltpu.VMEM((rows_pc, width), jnp.int32),
            pltpu.VMEM((rows_pc, width), jnp.int32),
            pltpu.SemaphoreType.DMA,
            pltpu.SemaphoreType.DMA,
        ],
    )
    def gather_kernel(table_hbm, idx_hbm, out_hbm, idx0, idx1, rows0, rows1,
                      sem0, sem1):
        wid = lax.axis_index("s") * NC + lax.axis_index("c")
        n_mine = (total_chunks - wid + NW - 1) // NW
        idx_b, rows_b, sem_b = (idx0, idx1), (rows0, rows1), (sem0, sem1)

        def start_gather(i, b):
            ci = wid + i * NW
            pltpu.sync_copy(
                idx_hbm.at[pl.ds(rows_pc * ci, rows_pc)], idx_b[b])
            pltpu.async_copy(table_hbm.at[idx_b[b]], rows_b[b], sem_b[b])

        def finish(i, b):
            ci = wid + i * NW
            pltpu.make_async_copy(table_hbm.at[idx_b[b]], rows_b[b],
                                  sem_b[b]).wait()
            pltpu.sync_copy(rows_b[b],
                            out_hbm.at[pl.ds(ci * rows_pc, rows_pc)])

        @pl.when(n_mine > 0)
        def _prime():
            start_gather(0, 0)

        def pair(p, carry):
            i0, i1 = 2 * p, 2 * p + 1

            @pl.when(i1 < n_mine)
            def _s1():
                start_gather(i1, 1)

            finish(i0, 0)

            @pl.when(i1 + 1 < n_mine)
            def _s0():
                start_gather(i1 + 1, 0)

            @pl.when(i1 < n_mine)
            def _f1():
                finish(i1, 1)

            return carry

        lax.fori_loop(0, (n_mine + 1) // 2, pair, 0)

    return gather_kernel(table, idx_flat)


def _proj_body(x_ref, w_ref, inp_ref, msg_ref):
    acc = jnp.dot(x_ref[...], w_ref[...], preferred_element_type=jnp.float32)
    inp_ref[...] = acc
    msg_ref[...] = _pack_bf16(jnp.maximum(acc, 0.0))


def _step_core(g_ref, inp_ref, wt_ref, wb_ref):
    m = inp_ref.shape[0]
    lo, hi = _unpack_f32(g_ref[...])
    lo_s = _sum_slots(lo, m)
    hi_s = _sum_slots(hi, m)
    return (inp_ref[...]
            + jnp.dot(lo_s, wt_ref[...], preferred_element_type=jnp.float32)
            + jnp.dot(hi_s, wb_ref[...], preferred_element_type=jnp.float32))


def _step_body(g_ref, inp_ref, wt_ref, wb_ref, buf_ref, msg_ref):
    del buf_ref
    h = _step_core(g_ref, inp_ref, wt_ref, wb_ref)
    msg_ref[...] = _pack_bf16(jnp.maximum(h, 0.0))


def _step_last_body(g_ref, inp_ref, wt_ref, wb_ref, buf_ref, h_ref):
    del buf_ref
    h_ref[...] = _pack_bf16(_step_core(g_ref, inp_ref, wt_ref, wb_ref))


def _tail_body(apm, af_ref, msgs_ref, gf_ref, woa_ref, womt_ref, womb_ref,
               bo_ref, w1g_ref, w1m_ref, b1_ref, w2_ref, b2_ref, w3t_ref,
               out_ref):
    n_atoms_blk = af_ref.shape[0]
    lo, hi = _unpack_f32(msgs_ref[...])
    lo_s = _sum_slots(lo, n_atoms_blk)
    hi_s = _sum_slots(hi, n_atoms_blk)
    hidden = jnp.maximum(
        jnp.dot(af_ref[...], woa_ref[...], preferred_element_type=jnp.float32)
        + jnp.dot(lo_s, womt_ref[...], preferred_element_type=jnp.float32)
        + jnp.dot(hi_s, womb_ref[...], preferred_element_type=jnp.float32)
        + bo_ref[...], 0.0)
    hid = hidden.shape[1]
    mols = n_atoms_blk // apm
    mol = jnp.mean(hidden.reshape(mols, apm, hid), axis=1)
    h1 = jnp.maximum(
        jnp.dot(mol, w1m_ref[...], preferred_element_type=jnp.float32)
        + jnp.dot(gf_ref[...], w1g_ref[...],
                  preferred_element_type=jnp.float32)
        + b1_ref[...], 0.0)
    h2 = jnp.maximum(
        jnp.dot(h1, w2_ref[...], preferred_element_type=jnp.float32)
        + b2_ref[...], 0.0)
    out_ref[...] = jnp.sum(h2 * w3t_ref[...], axis=1, keepdims=True)


def kernel(atom_features, f_ini_atoms_bonds, atom_to_incoming_bonds, mapping,
           global_features, W_i, W_h, W_o, b_o, W_ffn1, b_ffn1, W_ffn2,
           b_ffn2, W_ffn3, b_ffn3):
    n_atoms, atom_f = atom_features.shape
    n_bonds, concat_f = f_ini_atoms_bonds.shape
    n_mols, gf_dim = global_features.shape
    hid = W_h.shape[0]
    half = hid // 2
    apm = n_atoms // n_mols

    mb = 800
    inp, message = pl.pallas_call(
        _proj_body,
        grid=(n_bonds // mb,),
        in_specs=[
            pl.BlockSpec((mb, concat_f), lambda i: (i, 0)),
            pl.BlockSpec((concat_f, hid), lambda i: (0, 0)),
        ],
        out_specs=[
            pl.BlockSpec((mb, hid), lambda i: (i, 0)),
            pl.BlockSpec((mb, half), lambda i: (i, 0)),
        ],
        out_shape=[
            jax.ShapeDtypeStruct((n_bonds, hid), jnp.float32),
            jax.ShapeDtypeStruct((n_bonds, half), jnp.int32),
        ],
    )(f_ini_atoms_bonds, W_i)

    sb = 800
    nblk = n_bonds // sb
    na, nb = nblk // 2, nblk - nblk // 2
    rows_half = na * sb * MAX_IN
    map_flat = _slot_major_idx(mapping, sb)
    map_a, map_b = map_flat[:rows_half], map_flat[rows_half:]
    wh_top, wh_bot = W_h[:half], W_h[half:]

    def make_step(body, off, nb_half):
        return pl.pallas_call(
            body,
            grid=(nb_half,),
            in_specs=[
                pl.BlockSpec((MAX_IN * sb, half), lambda i: (i, 0)),
                pl.BlockSpec((sb, hid), lambda i: (i + off, 0)),
                pl.BlockSpec((half, hid), lambda i: (0, 0)),
                pl.BlockSpec((half, hid), lambda i: (0, 0)),
                pl.BlockSpec(memory_space=pltpu.MemorySpace.HBM),
            ],
            out_specs=pl.BlockSpec((sb, half), lambda i: (i + off, 0)),
            out_shape=jax.ShapeDtypeStruct((n_bonds, half), jnp.int32),
            input_output_aliases={4: 0},
        )

    step_a = make_step(_step_body, 0, na)
    step_b = make_step(_step_body, na, nb)
    last_a = make_step(_step_last_body, 0, na)
    last_b = make_step(_step_last_body, na, nb)
    hist = [message]
    for d in range(1, DEPTH - 1):
        cur = hist[-1]
        ga = _gather_sc(cur, map_a)
        gb = _gather_sc(cur, map_b)
        donor = (jnp.zeros((n_bonds, half), jnp.int32) if d == 1
                 else hist[d - 2])
        buf_a = step_a(ga, inp, wh_top, wh_bot, donor)
        hist.append(step_b(gb, inp, wh_top, wh_bot, buf_a))
    cur = hist[-1]
    ga = _gather_sc(cur, map_a)
    gb = _gather_sc(cur, map_b)
    buf_a = last_a(ga, inp, wh_top, wh_bot, hist[DEPTH - 3])
    h_message = last_b(gb, inp, wh_top, wh_bot, buf_a)

    mol_blk = 40
    atoms_blk = mol_blk * apm
    atib_flat = _slot_major_idx(atom_to_incoming_bonds, atoms_blk)
    msgs_to_atoms = _gather_sc(h_message, atib_flat, rows_pc=80)

    w_om = W_o[atom_f:]
    out = pl.pallas_call(
        functools.partial(_tail_body, apm),
        grid=(n_mols // mol_blk,),
        in_specs=[
            pl.BlockSpec((atoms_blk, atom_f), lambda i: (i, 0)),
            pl.BlockSpec((MAX_IN * atoms_blk, half), lambda i: (i, 0)),
            pl.BlockSpec((mol_blk, gf_dim), lambda i: (i, 0)),
            pl.BlockSpec((atom_f, hid), lambda i: (0, 0)),
            pl.BlockSpec((half, hid), lambda i: (0, 0)),
            pl.BlockSpec((half, hid), lambda i: (0, 0)),
            pl.BlockSpec((1, hid), lambda i: (0, 0)),
            pl.BlockSpec((gf_dim, hid), lambda i: (0, 0)),
            pl.BlockSpec((hid, hid), lambda i: (0, 0)),
            pl.BlockSpec((1, hid), lambda i: (0, 0)),
            pl.BlockSpec((hid, hid), lambda i: (0, 0)),
            pl.BlockSpec((1, hid), lambda i: (0, 0)),
            pl.BlockSpec((1, hid), lambda i: (0, 0)),
        ],
        out_specs=pl.BlockSpec((mol_blk, 1), lambda i: (i, 0)),
        out_shape=jax.ShapeDtypeStruct((n_mols, 1), jnp.float32),
    )(atom_features, msgs_to_atoms, global_features,
      W_o[:atom_f], w_om[:half], w_om[half:], b_o.reshape(1, hid),
      W_ffn1[hid:], W_ffn1[:hid], b_ffn1.reshape(1, hid),
      W_ffn2, b_ffn2.reshape(1, hid), W_ffn3.reshape(1, hid))
    return out + b_ffn3


# speedup vs baseline: 1.0328x; 1.0312x over previous
import functools

import jax
import jax.numpy as jnp
from jax import lax
from jax.experimental import pallas as pl
from jax.experimental.pallas import tpu as pltpu
from jax.experimental.pallas import tpu_sc as plsc

DEPTH = 5
NC, NS = 2, 16
NW = NC * NS
MAX_IN = 4
_MASK_HI = -65536


def _pack_bf16(x):
    half = x.shape[1] // 2
    xb = x.astype(jnp.bfloat16)
    lo = lax.convert_element_type(
        lax.bitcast_convert_type(xb[:, :half], jnp.uint16), jnp.uint32)
    hi = lax.convert_element_type(
        lax.bitcast_convert_type(xb[:, half:], jnp.uint16), jnp.uint32)
    return lax.bitcast_convert_type(lo | (hi << 16), jnp.int32)


def _unpack_f32(w):
    lo = lax.bitcast_convert_type(w << 16, jnp.float32)
    hi = lax.bitcast_convert_type(w & _MASK_HI, jnp.float32)
    return lo, hi


def _sum_slots(g, m):
    return ((g[0 * m:1 * m] + g[1 * m:2 * m])
            + (g[2 * m:3 * m] + g[3 * m:4 * m]))


def _slot_major_idx(idx2d, blk):
    m = idx2d.shape[0]
    return (idx2d.reshape(m // blk, blk, MAX_IN)
            .transpose(0, 2, 1).reshape(-1).astype(jnp.int32))


def _gather_sc(table, idx_flat, rows_pc=128):
    n_rows, width = table.shape
    total_rows = idx_flat.shape[0]
    total_chunks = total_rows // rows_pc
    assert total_rows % rows_pc == 0 and rows_pc <= 128
    mesh = plsc.VectorSubcoreMesh(core_axis_name="c", subcore_axis_name="s",
                                  num_cores=NC, num_subcores=NS)

    @functools.partial(
        pl.kernel,
        out_type=jax.ShapeDtypeStruct((total_rows, width), jnp.int32),
        mesh=mesh,
        scratch_types=[
            pltpu.VMEM((rows_pc,), jnp.int32),
            pltpu.VMEM((rows_pc,), jnp.int32),
            pltpu.VMEM((rows_pc, width), jnp.int32),
            pltpu.VMEM((rows_pc, width), jnp.int32),
            pltpu.SemaphoreType.DMA,
            pltpu.SemaphoreType.DMA,
        ],
    )
    def gather_kernel(table_hbm, idx_hbm, out_hbm, idx0, idx1, rows0, rows1,
                      sem0, sem1):
        wid = lax.axis_index("s") * NC + lax.axis_index("c")
        n_mine = (total_chunks - wid + NW - 1) // NW
        idx_b, rows_b, sem_b = (idx0, idx1), (rows0, rows1), (sem0, sem1)

        def start_gather(i, b):
            ci = wid + i * NW
            pltpu.sync_copy(
                idx_hbm.at[pl.ds(rows_pc * ci, rows_pc)], idx_b[b])
            pltpu.async_copy(table_hbm.at[idx_b[b]], rows_b[b], sem_b[b])

        def finish(i, b):
            ci = wid + i * NW
            pltpu.make_async_copy(table_hbm.at[idx_b[b]], rows_b[b],
                                  sem_b[b]).wait()
            pltpu.sync_copy(rows_b[b],
                            out_hbm.at[pl.ds(ci * rows_pc, rows_pc)])

        @pl.when(n_mine > 0)
        def _prime():
            start_gather(0, 0)

        def pair(p, carry):
            i0, i1 = 2 * p, 2 * p + 1

            @pl.when(i1 < n_mine)
            def _s1():
                start_gather(i1, 1)

            finish(i0, 0)

            @pl.when(i1 + 1 < n_mine)
            def _s0():
                start_gather(i1 + 1, 0)

            @pl.when(i1 < n_mine)
            def _f1():
                finish(i1, 1)

            return carry

        lax.fori_loop(0, (n_mine + 1) // 2, pair, 0)

    return gather_kernel(table, idx_flat)


def _proj_body(x_ref, w_ref, inp_ref, msg_ref):
    acc = jnp.dot(x_ref[...], w_ref[...], preferred_element_type=jnp.float32)
    inp_ref[...] = _pack_bf16(acc)
    msg_ref[...] = _pack_bf16(jnp.maximum(acc, 0.0))


def _step_core(g_ref, inp_ref, wt_ref, wb_ref):
    m = inp_ref.shape[0]
    lo, hi = _unpack_f32(g_ref[...])
    lo_s = _sum_slots(lo, m)
    hi_s = _sum_slots(hi, m)
    ilo, ihi = _unpack_f32(inp_ref[...])
    return (jnp.concatenate([ilo, ihi], axis=1)
            + jnp.dot(lo_s, wt_ref[...], preferred_element_type=jnp.float32)
            + jnp.dot(hi_s, wb_ref[...], preferred_element_type=jnp.float32))


def _step_body(g_ref, inp_ref, wt_ref, wb_ref, buf_ref, msg_ref):
    del buf_ref
    h = _step_core(g_ref, inp_ref, wt_ref, wb_ref)
    msg_ref[...] = _pack_bf16(jnp.maximum(h, 0.0))


def _step_last_body(g_ref, inp_ref, wt_ref, wb_ref, buf_ref, h_ref):
    del buf_ref
    h_ref[...] = _pack_bf16(_step_core(g_ref, inp_ref, wt_ref, wb_ref))


def _tail_body(apm, af_ref, msgs_ref, gf_ref, woa_ref, womt_ref, womb_ref,
               bo_ref, w1g_ref, w1m_ref, b1_ref, w2_ref, b2_ref, w3t_ref,
               out_ref):
    n_atoms_blk = af_ref.shape[0]
    lo, hi = _unpack_f32(msgs_ref[...])
    lo_s = _sum_slots(lo, n_atoms_blk)
    hi_s = _sum_slots(hi, n_atoms_blk)
    hidden = jnp.maximum(
        jnp.dot(af_ref[...], woa_ref[...], preferred_element_type=jnp.float32)
        + jnp.dot(lo_s, womt_ref[...], preferred_element_type=jnp.float32)
        + jnp.dot(hi_s, womb_ref[...], preferred_element_type=jnp.float32)
        + bo_ref[...], 0.0)
    hid = hidden.shape[1]
    mols = n_atoms_blk // apm
    mol = jnp.mean(hidden.reshape(mols, apm, hid), axis=1)
    h1 = jnp.maximum(
        jnp.dot(mol, w1m_ref[...], preferred_element_type=jnp.float32)
        + jnp.dot(gf_ref[...], w1g_ref[...],
                  preferred_element_type=jnp.float32)
        + b1_ref[...], 0.0)
    h2 = jnp.maximum(
        jnp.dot(h1, w2_ref[...], preferred_element_type=jnp.float32)
        + b2_ref[...], 0.0)
    out_ref[...] = jnp.sum(h2 * w3t_ref[...], axis=1, keepdims=True)


def kernel(atom_features, f_ini_atoms_bonds, atom_to_incoming_bonds, mapping,
           global_features, W_i, W_h, W_o, b_o, W_ffn1, b_ffn1, W_ffn2,
           b_ffn2, W_ffn3, b_ffn3):
    n_atoms, atom_f = atom_features.shape
    n_bonds, concat_f = f_ini_atoms_bonds.shape
    n_mols, gf_dim = global_features.shape
    hid = W_h.shape[0]
    half = hid // 2
    apm = n_atoms // n_mols

    mb = 800
    inp, message = pl.pallas_call(
        _proj_body,
        grid=(n_bonds // mb,),
        in_specs=[
            pl.BlockSpec((mb, concat_f), lambda i: (i, 0)),
            pl.BlockSpec((concat_f, hid), lambda i: (0, 0)),
        ],
        out_specs=[
            pl.BlockSpec((mb, half), lambda i: (i, 0)),
            pl.BlockSpec((mb, half), lambda i: (i, 0)),
        ],
        out_shape=[
            jax.ShapeDtypeStruct((n_bonds, half), jnp.int32),
            jax.ShapeDtypeStruct((n_bonds, half), jnp.int32),
        ],
    )(f_ini_atoms_bonds, W_i)

    sb = 800
    nblk = n_bonds // sb
    na, nb = nblk // 2, nblk - nblk // 2
    rows_half = na * sb * MAX_IN
    map_flat = _slot_major_idx(mapping, sb)
    map_a, map_b = map_flat[:rows_half], map_flat[rows_half:]
    wh_top, wh_bot = W_h[:half], W_h[half:]

    def make_step(body, off, nb_half):
        return pl.pallas_call(
            body,
            grid=(nb_half,),
            in_specs=[
                pl.BlockSpec((MAX_IN * sb, half), lambda i: (i, 0)),
                pl.BlockSpec((sb, half), lambda i: (i + off, 0)),
                pl.BlockSpec((half, hid), lambda i: (0, 0)),
                pl.BlockSpec((half, hid), lambda i: (0, 0)),
                pl.BlockSpec(memory_space=pltpu.MemorySpace.HBM),
            ],
            out_specs=pl.BlockSpec((sb, half), lambda i: (i + off, 0)),
            out_shape=jax.ShapeDtypeStruct((n_bonds, half), jnp.int32),
            input_output_aliases={4: 0},
        )

    step_a = make_step(_step_body, 0, na)
    step_b = make_step(_step_body, na, nb)
    last_a = make_step(_step_last_body, 0, na)
    last_b = make_step(_step_last_body, na, nb)
    hist = [message]
    for d in range(1, DEPTH - 1):
        cur = hist[-1]
        ga = _gather_sc(cur, map_a)
        gb = _gather_sc(cur, map_b)
        donor = (jnp.zeros((n_bonds, half), jnp.int32) if d == 1
                 else hist[d - 2])
        buf_a = step_a(ga, inp, wh_top, wh_bot, donor)
        hist.append(step_b(gb, inp, wh_top, wh_bot, buf_a))
    cur = hist[-1]
    ga = _gather_sc(cur, map_a)
    gb = _gather_sc(cur, map_b)
    buf_a = last_a(ga, inp, wh_top, wh_bot, hist[DEPTH - 3])
    h_message = last_b(gb, inp, wh_top, wh_bot, buf_a)

    mol_blk = 40
    atoms_blk = mol_blk * apm
    atib_flat = _slot_major_idx(atom_to_incoming_bonds, atoms_blk)
    msgs_to_atoms = _gather_sc(h_message, atib_flat, rows_pc=80)

    w_om = W_o[atom_f:]
    out = pl.pallas_call(
        functools.partial(_tail_body, apm),
        grid=(n_mols // mol_blk,),
        in_specs=[
            pl.BlockSpec((atoms_blk, atom_f), lambda i: (i, 0)),
            pl.BlockSpec((MAX_IN * atoms_blk, half), lambda i: (i, 0)),
            pl.BlockSpec((mol_blk, gf_dim), lambda i: (i, 0)),
            pl.BlockSpec((atom_f, hid), lambda i: (0, 0)),
            pl.BlockSpec((half, hid), lambda i: (0, 0)),
            pl.BlockSpec((half, hid), lambda i: (0, 0)),
            pl.BlockSpec((1, hid), lambda i: (0, 0)),
            pl.BlockSpec((gf_dim, hid), lambda i: (0, 0)),
            pl.BlockSpec((hid, hid), lambda i: (0, 0)),
            pl.BlockSpec((1, hid), lambda i: (0, 0)),
            pl.BlockSpec((hid, hid), lambda i: (0, 0)),
            pl.BlockSpec((1, hid), lambda i: (0, 0)),
            pl.BlockSpec((1, hid), lambda i: (0, 0)),
        ],
        out_specs=pl.BlockSpec((mol_blk, 1), lambda i: (i, 0)),
        out_shape=jax.ShapeDtypeStruct((n_mols, 1), jnp.float32),
    )(atom_features, msgs_to_atoms, global_features,
      W_o[:atom_f], w_om[:half], w_om[half:], b_o.reshape(1, hid),
      W_ffn1[hid:], W_ffn1[:hid], b_ffn1.reshape(1, hid),
      W_ffn2, b_ffn2.reshape(1, hid), W_ffn3.reshape(1, hid))
    return out + b_ffn3


# speedup vs baseline: 1.0339x; 1.0011x over previous
import functools

import jax
import jax.numpy as jnp
from jax import lax
from jax.experimental import pallas as pl
from jax.experimental.pallas import tpu as pltpu
from jax.experimental.pallas import tpu_sc as plsc

DEPTH = 5
NC, NS = 2, 16
NW = NC * NS
MAX_IN = 4
_MASK_HI = -65536


def _pack_bf16(x):
    half = x.shape[1] // 2
    xb = x.astype(jnp.bfloat16)
    lo = lax.convert_element_type(
        lax.bitcast_convert_type(xb[:, :half], jnp.uint16), jnp.uint32)
    hi = lax.convert_element_type(
        lax.bitcast_convert_type(xb[:, half:], jnp.uint16), jnp.uint32)
    return lax.bitcast_convert_type(lo | (hi << 16), jnp.int32)


def _unpack_f32(w):
    lo = lax.bitcast_convert_type(w << 16, jnp.float32)
    hi = lax.bitcast_convert_type(w & _MASK_HI, jnp.float32)
    return lo, hi


def _sum_slots(g, m):
    return ((g[0 * m:1 * m] + g[1 * m:2 * m])
            + (g[2 * m:3 * m] + g[3 * m:4 * m]))


def _slot_major_idx(idx2d, blk):
    m = idx2d.shape[0]
    return (idx2d.reshape(m // blk, blk, MAX_IN)
            .transpose(0, 2, 1).reshape(-1).astype(jnp.int32))


def _gather_sc(table, idx_flat, rows_pc=128):
    n_rows, width = table.shape
    total_rows = idx_flat.shape[0]
    total_chunks = total_rows // rows_pc
    assert total_rows % rows_pc == 0 and rows_pc <= 128
    mesh = plsc.VectorSubcoreMesh(core_axis_name="c", subcore_axis_name="s",
                                  num_cores=NC, num_subcores=NS)

    @functools.partial(
        pl.kernel,
        out_type=jax.ShapeDtypeStruct((total_rows, width), jnp.int32),
        mesh=mesh,
        scratch_types=[
            pltpu.VMEM((rows_pc,), jnp.int32),
            pltpu.VMEM((rows_pc,), jnp.int32),
            pltpu.VMEM((rows_pc, width), jnp.int32),
            pltpu.VMEM((rows_pc, width), jnp.int32),
            pltpu.SemaphoreType.DMA,
            pltpu.SemaphoreType.DMA,
        ],
    )
    def gather_kernel(table_hbm, idx_hbm, out_hbm, idx0, idx1, rows0, rows1,
                      sem0, sem1):
        wid = lax.axis_index("s") * NC + lax.axis_index("c")
        n_mine = (total_chunks - wid + NW - 1) // NW
        idx_b, rows_b, sem_b = (idx0, idx1), (rows0, rows1), (sem0, sem1)

        def start_gather(i, b):
            ci = wid + i * NW
            pltpu.sync_copy(
                idx_hbm.at[pl.ds(rows_pc * ci, rows_pc)], idx_b[b])
            pltpu.async_copy(table_hbm.at[idx_b[b]], rows_b[b], sem_b[b])

        def finish(i, b):
            ci = wid + i * NW
            pltpu.make_async_copy(table_hbm.at[idx_b[b]], rows_b[b],
                                  sem_b[b]).wait()
            pltpu.sync_copy(rows_b[b],
                            out_hbm.at[pl.ds(ci * rows_pc, rows_pc)])

        @pl.when(n_mine > 0)
        def _prime():
            start_gather(0, 0)

        def pair(p, carry):
            i0, i1 = 2 * p, 2 * p + 1

            @pl.when(i1 < n_mine)
            def _s1():
                start_gather(i1, 1)

            finish(i0, 0)

            @pl.when(i1 + 1 < n_mine)
            def _s0():
                start_gather(i1 + 1, 0)

            @pl.when(i1 < n_mine)
            def _f1():
                finish(i1, 1)

            return carry

        lax.fori_loop(0, (n_mine + 1) // 2, pair, 0)

    return gather_kernel(table, idx_flat)


def _proj_body(x_ref, w_ref, inp_ref, msg_ref):
    acc = jnp.dot(x_ref[...], w_ref[...], preferred_element_type=jnp.float32)
    inp_ref[...] = _pack_bf16(acc)
    msg_ref[...] = _pack_bf16(jnp.maximum(acc, 0.0))


def _step_core(g_ref, inp_ref, wt_ref, wb_ref):
    m = inp_ref.shape[0]
    lo, hi = _unpack_f32(g_ref[...])
    lo_s = _sum_slots(lo, m)
    hi_s = _sum_slots(hi, m)
    ilo, ihi = _unpack_f32(inp_ref[...])
    return (jnp.concatenate([ilo, ihi], axis=1)
            + jnp.dot(lo_s, wt_ref[...], preferred_element_type=jnp.float32)
            + jnp.dot(hi_s, wb_ref[...], preferred_element_type=jnp.float32))


def _step_body(g_ref, inp_ref, wt_ref, wb_ref, buf_ref, msg_ref):
    del buf_ref
    h = _step_core(g_ref, inp_ref, wt_ref, wb_ref)
    msg_ref[...] = _pack_bf16(jnp.maximum(h, 0.0))


def _step_last_body(g_ref, inp_ref, wt_ref, wb_ref, buf_ref, h_ref):
    del buf_ref
    h_ref[...] = _pack_bf16(_step_core(g_ref, inp_ref, wt_ref, wb_ref))


def _tail_body(apm, af_ref, msgs_ref, gf_ref, woa_ref, womt_ref, womb_ref,
               bo_ref, w1g_ref, w1m_ref, b1_ref, w2_ref, b2_ref, w3t_ref,
               buf_ref, out_ref):
    del buf_ref
    n_atoms_blk = af_ref.shape[0]
    lo, hi = _unpack_f32(msgs_ref[...])
    lo_s = _sum_slots(lo, n_atoms_blk)
    hi_s = _sum_slots(hi, n_atoms_blk)
    hidden = jnp.maximum(
        jnp.dot(af_ref[...], woa_ref[...], preferred_element_type=jnp.float32)
        + jnp.dot(lo_s, womt_ref[...], preferred_element_type=jnp.float32)
        + jnp.dot(hi_s, womb_ref[...], preferred_element_type=jnp.float32)
        + bo_ref[...], 0.0)
    hid = hidden.shape[1]
    mols = n_atoms_blk // apm
    mol = jnp.mean(hidden.reshape(mols, apm, hid), axis=1)
    h1 = jnp.maximum(
        jnp.dot(mol, w1m_ref[...], preferred_element_type=jnp.float32)
        + jnp.dot(gf_ref[...], w1g_ref[...],
                  preferred_element_type=jnp.float32)
        + b1_ref[...], 0.0)
    h2 = jnp.maximum(
        jnp.dot(h1, w2_ref[...], preferred_element_type=jnp.float32)
        + b2_ref[...], 0.0)
    out_ref[...] = jnp.sum(h2 * w3t_ref[...], axis=1, keepdims=True)


def kernel(atom_features, f_ini_atoms_bonds, atom_to_incoming_bonds, mapping,
           global_features, W_i, W_h, W_o, b_o, W_ffn1, b_ffn1, W_ffn2,
           b_ffn2, W_ffn3, b_ffn3):
    n_atoms, atom_f = atom_features.shape
    n_bonds, concat_f = f_ini_atoms_bonds.shape
    n_mols, gf_dim = global_features.shape
    hid = W_h.shape[0]
    half = hid // 2
    apm = n_atoms // n_mols

    mb = 800
    inp, message = pl.pallas_call(
        _proj_body,
        grid=(n_bonds // mb,),
        in_specs=[
            pl.BlockSpec((mb, concat_f), lambda i: (i, 0)),
            pl.BlockSpec((concat_f, hid), lambda i: (0, 0)),
        ],
        out_specs=[
            pl.BlockSpec((mb, half), lambda i: (i, 0)),
            pl.BlockSpec((mb, half), lambda i: (i, 0)),
        ],
        out_shape=[
            jax.ShapeDtypeStruct((n_bonds, half), jnp.int32),
            jax.ShapeDtypeStruct((n_bonds, half), jnp.int32),
        ],
    )(f_ini_atoms_bonds, W_i)

    sb = 800
    nblk = n_bonds // sb
    na, nb = nblk // 2, nblk - nblk // 2
    rows_half = na * sb * MAX_IN
    map_flat = _slot_major_idx(mapping, sb)
    map_a, map_b = map_flat[:rows_half], map_flat[rows_half:]
    wh_top, wh_bot = W_h[:half], W_h[half:]

    def make_step(body, off, nb_half):
        return pl.pallas_call(
            body,
            grid=(nb_half,),
            in_specs=[
                pl.BlockSpec((MAX_IN * sb, half), lambda i: (i, 0)),
                pl.BlockSpec((sb, half), lambda i: (i + off, 0)),
                pl.BlockSpec((half, hid), lambda i: (0, 0)),
                pl.BlockSpec((half, hid), lambda i: (0, 0)),
                pl.BlockSpec(memory_space=pltpu.MemorySpace.HBM),
            ],
            out_specs=pl.BlockSpec((sb, half), lambda i: (i + off, 0)),
            out_shape=jax.ShapeDtypeStruct((n_bonds, half), jnp.int32),
            input_output_aliases={4: 0},
        )

    step_a = make_step(_step_body, 0, na)
    step_b = make_step(_step_body, na, nb)
    last_a = make_step(_step_last_body, 0, na)
    last_b = make_step(_step_last_body, na, nb)
    hist = [message]
    for d in range(1, DEPTH - 1):
        cur = hist[-1]
        ga = _gather_sc(cur, map_a)
        gb = _gather_sc(cur, map_b)
        donor = (jnp.zeros((n_bonds, half), jnp.int32) if d == 1
                 else hist[d - 2])
        buf_a = step_a(ga, inp, wh_top, wh_bot, donor)
        hist.append(step_b(gb, inp, wh_top, wh_bot, buf_a))
    cur = hist[-1]
    ga = _gather_sc(cur, map_a)
    gb = _gather_sc(cur, map_b)
    buf_a = last_a(ga, inp, wh_top, wh_bot, hist[DEPTH - 3])
    h_message = last_b(gb, inp, wh_top, wh_bot, buf_a)

    mol_blk = 40
    n_mblk = n_mols // mol_blk
    ta, tb = n_mblk // 2, n_mblk - n_mblk // 2
    atoms_blk = mol_blk * apm
    atib_flat = _slot_major_idx(atom_to_incoming_bonds, atoms_blk)
    arows_half = ta * atoms_blk * MAX_IN
    m2a_a = _gather_sc(h_message, atib_flat[:arows_half], rows_pc=80)
    m2a_b = _gather_sc(h_message, atib_flat[arows_half:], rows_pc=80)

    def make_tail(off, nblk_t):
        return pl.pallas_call(
            functools.partial(_tail_body, apm),
            grid=(nblk_t,),
            in_specs=[
                pl.BlockSpec((atoms_blk, atom_f), lambda i: (i + off, 0)),
                pl.BlockSpec((MAX_IN * atoms_blk, half), lambda i: (i, 0)),
                pl.BlockSpec((mol_blk, gf_dim), lambda i: (i + off, 0)),
                pl.BlockSpec((atom_f, hid), lambda i: (0, 0)),
                pl.BlockSpec((half, hid), lambda i: (0, 0)),
                pl.BlockSpec((half, hid), lambda i: (0, 0)),
                pl.BlockSpec((1, hid), lambda i: (0, 0)),
                pl.BlockSpec((gf_dim, hid), lambda i: (0, 0)),
                pl.BlockSpec((hid, hid), lambda i: (0, 0)),
                pl.BlockSpec((1, hid), lambda i: (0, 0)),
                pl.BlockSpec((hid, hid), lambda i: (0, 0)),
                pl.BlockSpec((1, hid), lambda i: (0, 0)),
                pl.BlockSpec((1, hid), lambda i: (0, 0)),
                pl.BlockSpec(memory_space=pltpu.MemorySpace.HBM),
            ],
            out_specs=pl.BlockSpec((mol_blk, 1), lambda i: (i + off, 0)),
            out_shape=jax.ShapeDtypeStruct((n_mols, 1), jnp.float32),
            input_output_aliases={13: 0},
        )

    w_om = W_o[atom_f:]
    wtail = (W_o[:atom_f], w_om[:half], w_om[half:], b_o.reshape(1, hid),
             W_ffn1[hid:], W_ffn1[:hid], b_ffn1.reshape(1, hid),
             W_ffn2, b_ffn2.reshape(1, hid), W_ffn3.reshape(1, hid))
    out_a = make_tail(0, ta)(atom_features, m2a_a, global_features, *wtail,
                             jnp.zeros((n_mols, 1), jnp.float32))
    out = make_tail(ta, tb)(atom_features, m2a_b, global_features, *wtail,
                            out_a)
    return out + b_ffn3
